# baseline (speedup 1.0000x reference)
import functools

import jax
import jax.numpy as jnp
import numpy as np
from jax import lax
from jax.experimental import pallas as pl
from jax.experimental.pallas import tpu as pltpu
from jax.experimental.pallas import tpu_sc as plsc

_NVARS = 99
_SHAPE = (8, 40320, _NVARS)
_TOTAL = _SHAPE[0] * _SHAPE[1] * _SHAPE[2]
_NW = 32
_PER_W = _TOTAL // _NW
_PERIOD = 99 * 16
_CHUNK_P = 15
_CHUNK = _PERIOD * _CHUNK_P
_NCHUNK = _PER_W // _CHUNK
_NTRIP = _NCHUNK // 3


def _affine_consts():
    scale = np.ones(_NVARS, dtype=np.float64)
    shift = np.zeros(_NVARS, dtype=np.float64)
    for i, m in zip((10, 11, 12), (11.0, 12.0, 13.0)):
        scale[i] = 1.0 / m
    i = np.arange(13, _NVARS).astype(np.float64)
    mu = (0.1 * i).astype(np.float32).astype(np.float64)
    sd = (1.0 + 0.01 * i).astype(np.float32).astype(np.float64)
    scale[13:] = 1.0 / sd
    shift[13:] = -(mu / sd)
    return scale.astype(np.float32), shift.astype(np.float32)


def _pattern_1584():
    scale, shift = _affine_consts()
    idx = np.arange(_PERIOD) % _NVARS
    return jnp.asarray(scale[idx]), jnp.asarray(shift[idx])


def _sc_norm(xf, scp, shp):
    mesh = plsc.VectorSubcoreMesh(core_axis_name="c", subcore_axis_name="s")

    @functools.partial(
        pl.kernel,
        mesh=mesh,
        out_type=jax.ShapeDtypeStruct((_TOTAL,), jnp.float32),
        scratch_types=[
            pltpu.VMEM((_CHUNK,), jnp.float32),
            pltpu.VMEM((_CHUNK,), jnp.float32),
            pltpu.VMEM((_CHUNK,), jnp.float32),
            pltpu.VMEM((_PERIOD,), jnp.float32),
            pltpu.VMEM((_PERIOD,), jnp.float32),
            pltpu.SemaphoreType.DMA,
            pltpu.SemaphoreType.DMA,
            pltpu.SemaphoreType.DMA,
            pltpu.SemaphoreType.DMA,
            pltpu.SemaphoreType.DMA,
            pltpu.SemaphoreType.DMA,
        ],
    )
    def k(x_hbm, scp_hbm, shp_hbm, out_hbm,
          b0, b1, b2, scv, shv, si0, si1, si2, so0, so1, so2):
        wid = lax.axis_index("s") * 2 + lax.axis_index("c")
        base = wid * _PER_W
        bufs = (b0, b1, b2)
        sis = (si0, si1, si2)
        sos = (so0, so1, so2)

        pltpu.sync_copy(scp_hbm, scv)
        pltpu.sync_copy(shp_hbm, shv)

        def in_slice(c):
            return x_hbm.at[pl.ds(base + c * _CHUNK, _CHUNK)]

        def out_slice(c):
            return out_hbm.at[pl.ds(base + c * _CHUNK, _CHUNK)]

        def compute(buf):
            for j in range(_NVARS):
                a = scv[pl.ds(j * 16, 16)]
                s = shv[pl.ds(j * 16, 16)]
                for p in range(_CHUNK_P):
                    sl = pl.ds(p * _PERIOD + j * 16, 16)
                    buf[sl] = buf[sl] * a + s

        pltpu.async_copy(in_slice(0), b0, si0)
        pltpu.async_copy(in_slice(1), b1, si1)

        def triple(t, carry):
            for b in range(3):
                c = t * 3 + b
                buf, si, so = bufs[b], sis[b], sos[b]
                pb = (b - 1) % 3
                pbuf, psi, pso = bufs[pb], sis[pb], sos[pb]

                pltpu.make_async_copy(in_slice(c), buf, si).wait()

                if b == 0:
                    @pl.when(t >= 1)
                    def _():
                        pltpu.make_async_copy(pbuf, out_slice(c - 1), pso).wait()

                    pltpu.async_copy(in_slice(c + 2), pbuf, psi)
                else:
                    pltpu.make_async_copy(pbuf, out_slice(c - 1), pso).wait()

                    @pl.when(t < _NTRIP - 1)
                    def _():
                        pltpu.async_copy(in_slice(c + 2), pbuf, psi)

                compute(buf)
                pltpu.async_copy(buf, out_slice(c), so)
            return carry

        lax.fori_loop(0, _NTRIP, triple, 0)
        pltpu.make_async_copy(b2, out_slice(_NCHUNK - 1), so2).wait()

    return k(xf, scp, shp)


@functools.partial(jax.jit)
def kernel(x):
    scp, shp = _pattern_1584()
    out = _sc_norm(x.reshape(_TOTAL), scp, shp)
    return out.reshape(_SHAPE)


# speedup vs baseline: 2.1612x; 2.1612x over previous
import functools

import jax
import jax.numpy as jnp
import numpy as np
from jax import lax
from jax.experimental import pallas as pl
from jax.experimental.pallas import tpu as pltpu
from jax.experimental.pallas import tpu_sc as plsc

_NVARS = 99
_SHAPE = (8, 40320, _NVARS)
_NW = 32
_WPP = 4
_ROWS_W = _SHAPE[1] // _WPP
_RCHUNK = 240
_NCHUNK = _ROWS_W // _RCHUNK
_NTRIP = _NCHUNK // 3
_TAIL = 83


def _affine_consts():
    scale = np.ones(_NVARS, dtype=np.float64)
    shift = np.zeros(_NVARS, dtype=np.float64)
    for i, m in zip((10, 11, 12), (11.0, 12.0, 13.0)):
        scale[i] = 1.0 / m
    i = np.arange(13, _NVARS).astype(np.float64)
    mu = (0.1 * i).astype(np.float32).astype(np.float64)
    sd = (1.0 + 0.01 * i).astype(np.float32).astype(np.float64)
    scale[13:] = 1.0 / sd
    shift[13:] = -(mu / sd)
    return jnp.asarray(scale.astype(np.float32)), jnp.asarray(shift.astype(np.float32))


def _sc_norm(x, scp, shp):
    mesh = plsc.VectorSubcoreMesh(core_axis_name="c", subcore_axis_name="s")

    @functools.partial(
        pl.kernel,
        mesh=mesh,
        out_type=jax.ShapeDtypeStruct(_SHAPE, jnp.float32),
        scratch_types=[
            pltpu.VMEM((_RCHUNK, _NVARS), jnp.float32),
            pltpu.VMEM((_RCHUNK, _NVARS), jnp.float32),
            pltpu.VMEM((_RCHUNK, _NVARS), jnp.float32),
            pltpu.VMEM((_NVARS,), jnp.float32),
            pltpu.VMEM((_NVARS,), jnp.float32),
            pltpu.SemaphoreType.DMA,
            pltpu.SemaphoreType.DMA,
            pltpu.SemaphoreType.DMA,
            pltpu.SemaphoreType.DMA,
            pltpu.SemaphoreType.DMA,
            pltpu.SemaphoreType.DMA,
        ],
        compiler_params=pltpu.CompilerParams(use_tc_tiling_on_sc=True),
    )
    def k(x_hbm, scp_hbm, shp_hbm, out_hbm,
          b0, b1, b2, scv, shv, si0, si1, si2, so0, so1, so2):
        wid = lax.axis_index("s") * 2 + lax.axis_index("c")
        d0 = wid // _WPP
        r0 = (wid % _WPP) * _ROWS_W
        bufs = (b0, b1, b2)
        sis = (si0, si1, si2)
        sos = (so0, so1, so2)

        pltpu.sync_copy(scp_hbm, scv)
        pltpu.sync_copy(shp_hbm, shv)

        starts = [16 * j for j in range(6)] + [_TAIL]
        avs = [scv[pl.ds(s, 16)] for s in starts]
        bvs = [shv[pl.ds(s, 16)] for s in starts]

        def in_slice(c):
            return x_hbm.at[d0, pl.ds(r0 + c * _RCHUNK, _RCHUNK), :]

        def out_slice(c):
            return out_hbm.at[d0, pl.ds(r0 + c * _RCHUNK, _RCHUNK), :]

        def compute(buf):
            def rows8(g, carry):
                base = g * 8
                for rr in range(8):
                    row = base + rr
                    vals = [buf[row, pl.ds(s, 16)] for s in starts]
                    for (s, v, a, b) in zip(starts, vals, avs, bvs):
                        buf[row, pl.ds(s, 16)] = v * a + b
                return carry

            lax.fori_loop(0, _RCHUNK // 8, rows8, 0)

        pltpu.async_copy(in_slice(0), b0, si0)
        pltpu.async_copy(in_slice(1), b1, si1)

        def triple(t, carry):
            for b in range(3):
                c = t * 3 + b
                buf, si, so = bufs[b], sis[b], sos[b]
                pb = (b - 1) % 3
                pbuf, psi, pso = bufs[pb], sis[pb], sos[pb]

                pltpu.make_async_copy(in_slice(c), buf, si).wait()

                if b == 0:
                    @pl.when(t >= 1)
                    def _():
                        pltpu.make_async_copy(pbuf, out_slice(c - 1), pso).wait()

                    pltpu.async_copy(in_slice(c + 2), pbuf, psi)
                else:
                    pltpu.make_async_copy(pbuf, out_slice(c - 1), pso).wait()

                    @pl.when(t < _NTRIP - 1)
                    def _():
                        pltpu.async_copy(in_slice(c + 2), pbuf, psi)

                compute(buf)
                pltpu.async_copy(buf, out_slice(c), so)
            return carry

        lax.fori_loop(0, _NTRIP, triple, 0)
        pltpu.make_async_copy(b2, out_slice(_NCHUNK - 1), so2).wait()

    return k(x, scp, shp)


@functools.partial(jax.jit)
def kernel(x):
    scp, shp = _affine_consts()
    return _sc_norm(x, scp, shp)


# speedup vs baseline: 2.1731x; 1.0055x over previous
import functools

import jax
import jax.numpy as jnp
import numpy as np
from jax import lax
from jax.experimental import pallas as pl
from jax.experimental.pallas import tpu as pltpu
from jax.experimental.pallas import tpu_sc as plsc

_NVARS = 99
_SHAPE = (8, 40320, _NVARS)
_NW = 32
_WPP = 4
_ROWS_W = _SHAPE[1] // _WPP
_RCHUNK = 240
_NCHUNK = _ROWS_W // _RCHUNK
_NTRIP = _NCHUNK // 3
_TAIL = 83


def _affine_consts():
    scale = np.ones(_NVARS, dtype=np.float64)
    shift = np.zeros(_NVARS, dtype=np.float64)
    for i, m in zip((10, 11, 12), (11.0, 12.0, 13.0)):
        scale[i] = 1.0 / m
    i = np.arange(13, _NVARS).astype(np.float64)
    mu = (0.1 * i).astype(np.float32).astype(np.float64)
    sd = (1.0 + 0.01 * i).astype(np.float32).astype(np.float64)
    scale[13:] = 1.0 / sd
    shift[13:] = -(mu / sd)
    return jnp.asarray(scale.astype(np.float32)), jnp.asarray(shift.astype(np.float32))


def _sc_norm(x, scp, shp):
    mesh = plsc.VectorSubcoreMesh(core_axis_name="c", subcore_axis_name="s")

    @functools.partial(
        pl.kernel,
        mesh=mesh,
        out_type=jax.ShapeDtypeStruct(_SHAPE, jnp.float32),
        scratch_types=[
            pltpu.VMEM((_RCHUNK, _NVARS), jnp.float32),
            pltpu.VMEM((_RCHUNK, _NVARS), jnp.float32),
            pltpu.VMEM((_RCHUNK, _NVARS), jnp.float32),
            pltpu.VMEM((_NVARS,), jnp.float32),
            pltpu.VMEM((_NVARS,), jnp.float32),
            pltpu.SemaphoreType.DMA,
            pltpu.SemaphoreType.DMA,
            pltpu.SemaphoreType.DMA,
            pltpu.SemaphoreType.DMA,
            pltpu.SemaphoreType.DMA,
            pltpu.SemaphoreType.DMA,
        ],
        compiler_params=pltpu.CompilerParams(use_tc_tiling_on_sc=True),
    )
    def k(x_hbm, scp_hbm, shp_hbm, out_hbm,
          b0, b1, b2, scv, shv, si0, si1, si2, so0, so1, so2):
        wid = lax.axis_index("s") * 2 + lax.axis_index("c")
        d0 = wid // _WPP
        r0 = (wid % _WPP) * _ROWS_W
        bufs = (b0, b1, b2)
        sis = (si0, si1, si2)
        sos = (so0, so1, so2)

        pltpu.sync_copy(scp_hbm, scv)
        pltpu.sync_copy(shp_hbm, shv)

        starts = [16 * j for j in range(6)] + [_TAIL]
        avs = [scv[pl.ds(s, 16)] for s in starts]
        bvs = [shv[pl.ds(s, 16)] for s in starts]

        def in_slice(c):
            return x_hbm.at[d0, pl.ds(r0 + c * _RCHUNK, _RCHUNK), :]

        def out_slice(c):
            return out_hbm.at[d0, pl.ds(r0 + c * _RCHUNK, _RCHUNK), :]

        def compute(buf):
            @plsc.parallel_loop(0, _RCHUNK, step=1, unroll=8)
            def _(row):
                vals = [buf[row, pl.ds(s, 16)] for s in starts]
                for (s, v, a, b) in zip(starts, vals, avs, bvs):
                    buf[row, pl.ds(s, 16)] = v * a + b

        pltpu.async_copy(in_slice(0), b0, si0)
        pltpu.async_copy(in_slice(1), b1, si1)

        def triple(t, carry):
            for b in range(3):
                c = t * 3 + b
                buf, si, so = bufs[b], sis[b], sos[b]
                pb = (b - 1) % 3
                pbuf, psi, pso = bufs[pb], sis[pb], sos[pb]

                pltpu.make_async_copy(in_slice(c), buf, si).wait()

                if b == 0:
                    @pl.when(t >= 1)
                    def _():
                        pltpu.make_async_copy(pbuf, out_slice(c - 1), pso).wait()

                    pltpu.async_copy(in_slice(c + 2), pbuf, psi)
                else:
                    pltpu.make_async_copy(pbuf, out_slice(c - 1), pso).wait()

                    @pl.when(t < _NTRIP - 1)
                    def _():
                        pltpu.async_copy(in_slice(c + 2), pbuf, psi)

                compute(buf)
                pltpu.async_copy(buf, out_slice(c), so)
            return carry

        lax.fori_loop(0, _NTRIP, triple, 0)
        pltpu.make_async_copy(b2, out_slice(_NCHUNK - 1), so2).wait()

    return k(x, scp, shp)


@functools.partial(jax.jit)
def kernel(x):
    scp, shp = _affine_consts()
    return _sc_norm(x, scp, shp)
